# baseline (speedup 1.0000x reference)
import functools

import jax
import jax.numpy as jnp
from jax import lax
from jax.experimental import pallas as pl
from jax.experimental.pallas import tpu as pltpu
from jax.experimental.pallas import tpu_sc as plsc

N = 10000
E = 320000
DF = 128
FS = 40
KH = 2
OUT = 64
G = 16

NC = 2
NS = 16
LN = 16
NW = NC * NS
EPW = E // NW
CH = 80
NCHUNK = EPW // CH
RPT = N // NS

BN = 2000
NB = N // BN


def _dense_body(x_ref, w_ref, b_ref, aw_ref, ab_ref, f_ref, a_ref):
    f = jnp.dot(x_ref[...], w_ref[...], preferred_element_type=jnp.float32,
                precision=lax.Precision.HIGHEST) + b_ref[...]
    f_ref[...] = f
    a_ref[...] = jnp.dot(f, aw_ref[...], preferred_element_type=jnp.float32,
                         precision=lax.Precision.HIGHEST) + ab_ref[...]


def _dense(x, wt, b2, awt, ab2):
    n, din = x.shape
    p = wt.shape[1]
    return pl.pallas_call(
        _dense_body,
        grid=(n // BN,),
        in_specs=[pl.BlockSpec((BN, din), lambda i: (i, 0)),
                  pl.BlockSpec((din, p), lambda i: (0, 0)),
                  pl.BlockSpec((1, p), lambda i: (0, 0)),
                  pl.BlockSpec((p, 2), lambda i: (0, 0)),
                  pl.BlockSpec((1, 2), lambda i: (0, 0))],
        out_specs=[pl.BlockSpec((BN, p), lambda i: (i, 0)),
                   pl.BlockSpec((BN, 2), lambda i: (i, 0))],
        out_shape=[jax.ShapeDtypeStruct((n, p), jnp.float32),
                   jax.ShapeDtypeStruct((n, 2), jnp.float32)],
    )(x, wt, b2, awt, ab2)


ZR = 25
NBUF = 4


@functools.lru_cache(maxsize=None)
def _edge_kernel(P):
    vpr = P // LN
    use_fsh = P <= 48
    mesh = plsc.VectorSubcoreMesh(core_axis_name="c", subcore_axis_name="s")

    @functools.partial(
        pl.kernel,
        out_type=jax.ShapeDtypeStruct((NC, N, P), jnp.float32),
        mesh=mesh,
        compiler_params=pltpu.CompilerParams(use_tc_tiling_on_sc=False,
                                             needs_layout_passes=False),
        scratch_types=[
            pltpu.VMEM_SHARED((N, P), jnp.float32),
            pltpu.VMEM_SHARED((N if use_fsh else 1, P), jnp.float32),
            pltpu.VMEM((ZR, P), jnp.float32),
            pltpu.VMEM((2 * N,), jnp.float32),
            pltpu.VMEM((LN,), jnp.float32),
            pltpu.VMEM((NCHUNK, CH), jnp.int32),
            pltpu.VMEM((NCHUNK, CH), jnp.int32),
            pltpu.VMEM((CH,), jnp.float32),
        ] + [pltpu.VMEM((CH, P), jnp.float32) for _ in range(NBUF)]
          + [pltpu.SemaphoreType.DMA for _ in range(NBUF + 1)],
    )
    def k(f_hbm, a12_hbm, row_hbm, col_hbm, gm_hbm, u_out,
          u_sh, f_sh, z_v, a12_v, gm_v, ridx, cidx, ex_v,
          rows0, rows1, rows2, rows3, g0, g1, g2, g3, isem):
        cid = lax.axis_index("c")
        sid = lax.axis_index("s")
        wid = sid * NC + cid

        def zrow(r, carry):
            for j in range(vpr):
                z_v[r, pl.ds(j * LN, LN)] = jnp.zeros((LN,), jnp.float32)
            return carry
        lax.fori_loop(0, ZR, zrow, 0)

        cp_r = pltpu.async_copy(row_hbm.at[wid], ridx, isem)
        cp_c = pltpu.async_copy(col_hbm.at[wid], cidx, isem)
        if use_fsh:
            cp_f = pltpu.async_copy(f_hbm.at[pl.ds(sid * RPT, RPT)],
                                    f_sh.at[pl.ds(sid * RPT, RPT)], isem)
        f_tab = f_sh if use_fsh else f_hbm
        for t in range(RPT // ZR):
            pltpu.sync_copy(z_v, u_sh.at[pl.ds(sid * RPT + t * ZR, ZR)])
        pltpu.sync_copy(a12_hbm, a12_v)
        pltpu.sync_copy(gm_hbm, gm_v)
        cp_r.wait()
        cp_c.wait()
        if use_fsh:
            cp_f.wait()
        plsc.subcore_barrier()

        gmv = gm_v[...]
        rows = (rows0, rows1, rows2, rows3)
        gsems = (g0, g1, g2, g3)

        for b in range(NBUF):
            pltpu.async_copy(f_tab.at[cidx.at[b]], rows[b], gsems[b])

        def process(c, b):
            rv = rows[b]
            gs = gsems[b]
            pltpu.make_async_copy(f_tab.at[cidx.at[c]], rv, gs).wait()
            for g in range(CH // LN):
                r16 = ridx[c, pl.ds(g * LN, LN)]
                c16 = cidx[c, pl.ds(g * LN, LN)]
                a1 = plsc.load_gather(a12_v, [r16 * 2])
                a2 = plsc.load_gather(a12_v, [c16 * 2 + 1])
                v = a1 + a2
                v = jnp.where(v >= 0.0, v, 0.01 * v)
                ex_v[pl.ds(g * LN, LN)] = jnp.exp(v - gmv)
            for g in range(CH // LN):
                ev = ex_v[pl.ds(g * LN, LN)]
                for rr in range(LN):
                    r = g * LN + rr
                    e = ev[rr]
                    for j in range(vpr):
                        rv[r, pl.ds(j * LN, LN)] = rv[r, pl.ds(j * LN, LN)] * e
            pltpu.sync_copy(rv, u_sh.at[ridx.at[c]], add=True)

            @pl.when(c + NBUF < NCHUNK)
            def _next():
                pltpu.async_copy(f_tab.at[cidx.at[c + NBUF]], rv, gs)

        def quad(t, carry):
            for b in range(NBUF):
                process(NBUF * t + b, b)
            return carry
        lax.fori_loop(0, NCHUNK // NBUF, quad, 0)
        process(NCHUNK - 1, 0)

        plsc.subcore_barrier()
        pltpu.sync_copy(u_sh.at[pl.ds(sid * RPT, RPT)],
                        u_out.at[cid, pl.ds(sid * RPT, RPT)])

    return k


def _final_body(h0_ref, h1_ref, h2_ref, b0_ref, b1_ref, b2_ref,
                wc_ref, bc_ref, out_ref, acc, c0, c1, c2):
    i = pl.program_id(0)

    @pl.when(i == 0)
    def _init():
        acc[...] = jnp.zeros_like(acc)
        c0[...] = jnp.zeros_like(c0)
        c1[...] = jnp.zeros_like(c1)
        c2[...] = jnp.zeros_like(c2)

    iota = lax.broadcasted_iota(jnp.int32, (G, BN), 0)
    for lvl, (h_ref, b_ref, c_ref) in enumerate(
            ((h0_ref, b0_ref, c0), (h1_ref, b1_ref, c1), (h2_ref, b2_ref, c2))):
        m = (b_ref[0] == iota).astype(jnp.float32)
        acc[:, lvl * OUT:(lvl + 1) * OUT] += jnp.dot(
            m, h_ref[...], preferred_element_type=jnp.float32,
            precision=lax.Precision.HIGHEST)
        c_ref[...] += jnp.broadcast_to(
            jnp.sum(m, axis=1, keepdims=True), c_ref.shape)

    @pl.when(i == NB - 1)
    def _fin():
        cnt0 = jnp.maximum(c0[:, 0:1], 1.0)
        cnt1 = jnp.maximum(c1[:, 0:1], 1.0)
        cnt2 = jnp.maximum(c2[:, 0:1], 1.0)
        x = jnp.concatenate([acc[:, 0:OUT] / cnt0,
                             acc[:, OUT:2 * OUT] / cnt1,
                             acc[:, 2 * OUT:3 * OUT] / cnt2], axis=1)
        logits = jnp.dot(x, wc_ref[...], preferred_element_type=jnp.float32,
                         precision=lax.Precision.HIGHEST) + bc_ref[...]
        mx = jnp.max(logits, axis=1, keepdims=True)
        ez = jnp.exp(logits - mx)
        out_ref[...] = ez / jnp.sum(ez, axis=1, keepdims=True)


def _final(h0, h1, h2, b0, b1, b2, wc, bc):
    b3 = [b.reshape(NB, 1, BN) for b in (b0, b1, b2)]
    return pl.pallas_call(
        _final_body,
        grid=(NB,),
        in_specs=[pl.BlockSpec((BN, OUT), lambda i: (i, 0)),
                  pl.BlockSpec((BN, OUT), lambda i: (i, 0)),
                  pl.BlockSpec((BN, OUT), lambda i: (i, 0)),
                  pl.BlockSpec((1, 1, BN), lambda i: (i, 0, 0)),
                  pl.BlockSpec((1, 1, BN), lambda i: (i, 0, 0)),
                  pl.BlockSpec((1, 1, BN), lambda i: (i, 0, 0)),
                  pl.BlockSpec((3 * OUT, OUT), lambda i: (0, 0)),
                  pl.BlockSpec((1, OUT), lambda i: (0, 0))],
        out_specs=pl.BlockSpec((G, OUT), lambda i: (0, 0)),
        out_shape=jax.ShapeDtypeStruct((G, OUT), jnp.float32),
        scratch_shapes=[pltpu.VMEM((G, 3 * OUT), jnp.float32),
                        pltpu.VMEM((G, 128), jnp.float32),
                        pltpu.VMEM((G, 128), jnp.float32),
                        pltpu.VMEM((G, 128), jnp.float32)],
    )(h0, h1, h2, *b3, wc, bc)


def _pad_params(p, din, d, P):
    wt = jnp.zeros((din, P), jnp.float32).at[:, :d].set(p["W"].T)
    b2 = jnp.zeros((1, P), jnp.float32).at[0, :d].set(p["b"]).at[0, d].set(1.0)
    awt = jnp.zeros((P, 2), jnp.float32)
    awt = awt.at[:d, 0].set(p["a1w"][0]).at[:d, 1].set(p["a2w"][0])
    ab2 = jnp.concatenate([p["a1b"], p["a2b"]]).reshape(1, 2)
    return wt, b2, awt, ab2


def _sat(feats, row, col, p, din, d, P):
    wt, b2, awt, ab2 = _pad_params(p, din, d, P)
    f_aug, a12 = _dense(feats, wt, b2, awt, ab2)
    amax = jnp.max(a12, axis=0)
    gm = jnp.maximum(amax[0] + amax[1], 0.0)
    gm_arr = jnp.full((LN,), gm, jnp.float32)
    u = _edge_kernel(P)(f_aug, a12.reshape(-1), row, col, gm_arr)
    us = u[0] + u[1]
    return us[:, :d] / jnp.maximum(us[:, d:d + 1], 1e-30)


def _level(X, idxs, lvl, params):
    d1 = FS // KH
    h1 = jax.nn.relu(jnp.concatenate(
        [_sat(X, idxs[h][0], idxs[h][1], params["l%ss1h%d" % (lvl, h)],
              DF, d1, 32) for h in range(KH)], axis=1))
    h2 = jax.nn.relu(jnp.concatenate(
        [_sat(h1, idxs[h][0], idxs[h][1], params["l%ss2h%d" % (lvl, h)],
              FS, d1, 32) for h in range(KH)], axis=1))
    h3 = sum(_sat(h2, idxs[h][0], idxs[h][1], params["l%ss3h%d" % (lvl, h)],
                  FS, OUT, 80) for h in range(KH)) / 2.0
    return h3


def kernel(params, X0, X1, X2, L0, L1u, L1d, L2, batch0, batch1, batch2):
    def _sh(a):
        return a.reshape(NW, NCHUNK, CH)
    e0 = (_sh(L0[0]), _sh(L0[1]))
    e1u = (_sh(L1u[0]), _sh(L1u[1]))
    e1d = (_sh(L1d[0]), _sh(L1d[1]))
    e2 = (_sh(L2[0]), _sh(L2[1]))
    h3_0 = _level(X0, [e0, e0], "0", params)
    h3_1 = _level(X1, [e1u, e1d], "1", params)
    h3_2 = _level(X2, [e2, e2], "2", params)
    wc = params["out"]["W"].T
    bc = params["out"]["b"].reshape(1, OUT)
    return _final(h3_0, h3_1, h3_2, batch0, batch1, batch2, wc, bc)


# speedup vs baseline: 1.0694x; 1.0694x over previous
import functools

import jax
import jax.numpy as jnp
from jax import lax
from jax.experimental import pallas as pl
from jax.experimental.pallas import tpu as pltpu
from jax.experimental.pallas import tpu_sc as plsc

N = 10000
E = 320000
DF = 128
FS = 40
KH = 2
OUT = 64
G = 16

NC = 2
NS = 16
LN = 16
NW = NC * NS
EPW = E // NW
CH = 80
NCHUNK = EPW // CH
RPT = N // NS

BN = 2000
NB = N // BN


def _dense_body(x_ref, w_ref, b_ref, aw_ref, ab_ref, f_ref, a_ref):
    f = jnp.dot(x_ref[...], w_ref[...], preferred_element_type=jnp.float32,
                precision=lax.Precision.HIGHEST) + b_ref[...]
    f_ref[...] = f
    a_ref[...] = jnp.dot(f, aw_ref[...], preferred_element_type=jnp.float32,
                         precision=lax.Precision.HIGHEST) + ab_ref[...]


def _dense(x, wt, b2, awt, ab2):
    n, din = x.shape
    p = wt.shape[1]
    return pl.pallas_call(
        _dense_body,
        grid=(n // BN,),
        in_specs=[pl.BlockSpec((BN, din), lambda i: (i, 0)),
                  pl.BlockSpec((din, p), lambda i: (0, 0)),
                  pl.BlockSpec((1, p), lambda i: (0, 0)),
                  pl.BlockSpec((p, 2), lambda i: (0, 0)),
                  pl.BlockSpec((1, 2), lambda i: (0, 0))],
        out_specs=[pl.BlockSpec((BN, p), lambda i: (i, 0)),
                   pl.BlockSpec((BN, 2), lambda i: (i, 0))],
        out_shape=[jax.ShapeDtypeStruct((n, p), jnp.float32),
                   jax.ShapeDtypeStruct((n, 2), jnp.float32)],
    )(x, wt, b2, awt, ab2)


ZR = 25
NBUF = 2


@functools.lru_cache(maxsize=None)
def _edge_kernel(P):
    vpr = P // LN
    use_fsh = P <= 48
    mesh = plsc.VectorSubcoreMesh(core_axis_name="c", subcore_axis_name="s")

    @functools.partial(
        pl.kernel,
        out_type=jax.ShapeDtypeStruct((NC, N, P), jnp.float32),
        mesh=mesh,
        compiler_params=pltpu.CompilerParams(use_tc_tiling_on_sc=False,
                                             needs_layout_passes=False),
        scratch_types=[
            pltpu.VMEM_SHARED((N, P), jnp.float32),
            pltpu.VMEM_SHARED((N if use_fsh else 1, P), jnp.float32),
            pltpu.VMEM((ZR, P), jnp.float32),
            pltpu.VMEM((2 * N,), jnp.float32),
            pltpu.VMEM((LN,), jnp.float32),
            pltpu.VMEM((NCHUNK, CH), jnp.int32),
            pltpu.VMEM((NCHUNK, CH), jnp.int32),
            pltpu.VMEM((CH,), jnp.float32),
        ] + [pltpu.VMEM((CH, P), jnp.float32) for _ in range(NBUF)]
          + [pltpu.SemaphoreType.DMA for _ in range(NBUF + 1)],
    )
    def k(f_hbm, a12_hbm, row_hbm, col_hbm, gm_hbm, u_out,
          u_sh, f_sh, z_v, a12_v, gm_v, ridx, cidx, ex_v,
          rows0, rows1, g0, g1, isem):
        cid = lax.axis_index("c")
        sid = lax.axis_index("s")
        wid = sid * NC + cid

        def zrow(r, carry):
            for j in range(vpr):
                z_v[r, pl.ds(j * LN, LN)] = jnp.zeros((LN,), jnp.float32)
            return carry
        lax.fori_loop(0, ZR, zrow, 0)

        cp_r = pltpu.async_copy(row_hbm.at[wid], ridx, isem)
        cp_c = pltpu.async_copy(col_hbm.at[wid], cidx, isem)
        if use_fsh:
            cp_f = pltpu.async_copy(f_hbm.at[pl.ds(sid * RPT, RPT)],
                                    f_sh.at[pl.ds(sid * RPT, RPT)], isem)
        f_tab = f_sh if use_fsh else f_hbm
        for t in range(RPT // ZR):
            pltpu.sync_copy(z_v, u_sh.at[pl.ds(sid * RPT + t * ZR, ZR)])
        pltpu.sync_copy(a12_hbm, a12_v)
        pltpu.sync_copy(gm_hbm, gm_v)
        cp_r.wait()
        cp_c.wait()
        if use_fsh:
            cp_f.wait()
        plsc.subcore_barrier()

        gmv = gm_v[...]
        rows = (rows0, rows1)
        gsems = (g0, g1)

        for b in range(NBUF):
            pltpu.async_copy(f_tab.at[cidx.at[b]], rows[b], gsems[b])

        def process(c, b):
            rv = rows[b]
            gs = gsems[b]
            pltpu.make_async_copy(f_tab.at[cidx.at[c]], rv, gs).wait()
            for g in range(CH // LN):
                r16 = ridx[c, pl.ds(g * LN, LN)]
                c16 = cidx[c, pl.ds(g * LN, LN)]
                a1 = plsc.load_gather(a12_v, [r16 * 2])
                a2 = plsc.load_gather(a12_v, [c16 * 2 + 1])
                v = a1 + a2
                v = jnp.where(v >= 0.0, v, 0.01 * v)
                ex_v[pl.ds(g * LN, LN)] = jnp.exp(v - gmv)
            for g in range(CH // LN):
                ev = ex_v[pl.ds(g * LN, LN)]
                for rr in range(LN):
                    r = g * LN + rr
                    e = ev[rr]
                    for j in range(vpr):
                        rv[r, pl.ds(j * LN, LN)] = rv[r, pl.ds(j * LN, LN)] * e
            pltpu.sync_copy(rv, u_sh.at[ridx.at[c]], add=True)

            @pl.when(c + NBUF < NCHUNK)
            def _next():
                pltpu.async_copy(f_tab.at[cidx.at[c + NBUF]], rv, gs)

        def quad(t, carry):
            for b in range(NBUF):
                process(NBUF * t + b, b)
            return carry
        lax.fori_loop(0, NCHUNK // NBUF, quad, 0)
        process(NCHUNK - 1, 0)

        plsc.subcore_barrier()
        pltpu.sync_copy(u_sh.at[pl.ds(sid * RPT, RPT)],
                        u_out.at[cid, pl.ds(sid * RPT, RPT)])

    return k


def _final_body(h0_ref, h1_ref, h2_ref, b0_ref, b1_ref, b2_ref,
                wc_ref, bc_ref, out_ref, acc, c0, c1, c2):
    i = pl.program_id(0)

    @pl.when(i == 0)
    def _init():
        acc[...] = jnp.zeros_like(acc)
        c0[...] = jnp.zeros_like(c0)
        c1[...] = jnp.zeros_like(c1)
        c2[...] = jnp.zeros_like(c2)

    iota = lax.broadcasted_iota(jnp.int32, (G, BN), 0)
    for lvl, (h_ref, b_ref, c_ref) in enumerate(
            ((h0_ref, b0_ref, c0), (h1_ref, b1_ref, c1), (h2_ref, b2_ref, c2))):
        m = (b_ref[0] == iota).astype(jnp.float32)
        acc[:, lvl * OUT:(lvl + 1) * OUT] += jnp.dot(
            m, h_ref[...], preferred_element_type=jnp.float32,
            precision=lax.Precision.HIGHEST)
        c_ref[...] += jnp.broadcast_to(
            jnp.sum(m, axis=1, keepdims=True), c_ref.shape)

    @pl.when(i == NB - 1)
    def _fin():
        cnt0 = jnp.maximum(c0[:, 0:1], 1.0)
        cnt1 = jnp.maximum(c1[:, 0:1], 1.0)
        cnt2 = jnp.maximum(c2[:, 0:1], 1.0)
        x = jnp.concatenate([acc[:, 0:OUT] / cnt0,
                             acc[:, OUT:2 * OUT] / cnt1,
                             acc[:, 2 * OUT:3 * OUT] / cnt2], axis=1)
        logits = jnp.dot(x, wc_ref[...], preferred_element_type=jnp.float32,
                         precision=lax.Precision.HIGHEST) + bc_ref[...]
        mx = jnp.max(logits, axis=1, keepdims=True)
        ez = jnp.exp(logits - mx)
        out_ref[...] = ez / jnp.sum(ez, axis=1, keepdims=True)


def _final(h0, h1, h2, b0, b1, b2, wc, bc):
    b3 = [b.reshape(NB, 1, BN) for b in (b0, b1, b2)]
    return pl.pallas_call(
        _final_body,
        grid=(NB,),
        in_specs=[pl.BlockSpec((BN, OUT), lambda i: (i, 0)),
                  pl.BlockSpec((BN, OUT), lambda i: (i, 0)),
                  pl.BlockSpec((BN, OUT), lambda i: (i, 0)),
                  pl.BlockSpec((1, 1, BN), lambda i: (i, 0, 0)),
                  pl.BlockSpec((1, 1, BN), lambda i: (i, 0, 0)),
                  pl.BlockSpec((1, 1, BN), lambda i: (i, 0, 0)),
                  pl.BlockSpec((3 * OUT, OUT), lambda i: (0, 0)),
                  pl.BlockSpec((1, OUT), lambda i: (0, 0))],
        out_specs=pl.BlockSpec((G, OUT), lambda i: (0, 0)),
        out_shape=jax.ShapeDtypeStruct((G, OUT), jnp.float32),
        scratch_shapes=[pltpu.VMEM((G, 3 * OUT), jnp.float32),
                        pltpu.VMEM((G, 128), jnp.float32),
                        pltpu.VMEM((G, 128), jnp.float32),
                        pltpu.VMEM((G, 128), jnp.float32)],
    )(h0, h1, h2, *b3, wc, bc)


def _pad_params(p, din, d, P):
    wt = jnp.zeros((din, P), jnp.float32).at[:, :d].set(p["W"].T)
    b2 = jnp.zeros((1, P), jnp.float32).at[0, :d].set(p["b"]).at[0, d].set(1.0)
    awt = jnp.zeros((P, 2), jnp.float32)
    awt = awt.at[:d, 0].set(p["a1w"][0]).at[:d, 1].set(p["a2w"][0])
    ab2 = jnp.concatenate([p["a1b"], p["a2b"]]).reshape(1, 2)
    return wt, b2, awt, ab2


def _sat(feats, row, col, p, din, d, P):
    wt, b2, awt, ab2 = _pad_params(p, din, d, P)
    f_aug, a12 = _dense(feats, wt, b2, awt, ab2)
    amax = jnp.max(a12, axis=0)
    gm = jnp.maximum(amax[0] + amax[1], 0.0)
    gm_arr = jnp.full((LN,), gm, jnp.float32)
    u = _edge_kernel(P)(f_aug, a12.reshape(-1), row, col, gm_arr)
    us = u[0] + u[1]
    return us[:, :d] / jnp.maximum(us[:, d:d + 1], 1e-30)


def _level(X, idxs, lvl, params):
    d1 = FS // KH
    h1 = jax.nn.relu(jnp.concatenate(
        [_sat(X, idxs[h][0], idxs[h][1], params["l%ss1h%d" % (lvl, h)],
              DF, d1, 32) for h in range(KH)], axis=1))
    h2 = jax.nn.relu(jnp.concatenate(
        [_sat(h1, idxs[h][0], idxs[h][1], params["l%ss2h%d" % (lvl, h)],
              FS, d1, 32) for h in range(KH)], axis=1))
    h3 = sum(_sat(h2, idxs[h][0], idxs[h][1], params["l%ss3h%d" % (lvl, h)],
                  FS, OUT, 80) for h in range(KH)) / 2.0
    return h3


def kernel(params, X0, X1, X2, L0, L1u, L1d, L2, batch0, batch1, batch2):
    def _sh(a):
        return a.reshape(NW, NCHUNK, CH)
    e0 = (_sh(L0[0]), _sh(L0[1]))
    e1u = (_sh(L1u[0]), _sh(L1u[1]))
    e1d = (_sh(L1d[0]), _sh(L1d[1]))
    e2 = (_sh(L2[0]), _sh(L2[1]))
    h3_0 = _level(X0, [e0, e0], "0", params)
    h3_1 = _level(X1, [e1u, e1d], "1", params)
    h3_2 = _level(X2, [e2, e2], "2", params)
    wc = params["out"]["W"].T
    bc = params["out"]["b"].reshape(1, OUT)
    return _final(h3_0, h3_1, h3_2, batch0, batch1, batch2, wc, bc)


# speedup vs baseline: 1.1056x; 1.0338x over previous
import functools

import jax
import jax.numpy as jnp
from jax import lax
from jax.experimental import pallas as pl
from jax.experimental.pallas import tpu as pltpu
from jax.experimental.pallas import tpu_sc as plsc

N = 10000
E = 320000
DF = 128
FS = 40
KH = 2
OUT = 64
G = 16

NC = 2
NS = 16
LN = 16
NW = NC * NS
EPW = E // NW
CH = 80
NCHUNK = EPW // CH
RPT = N // NS

BN = 2000
NB = N // BN


def _dense_body(x_ref, w_ref, b_ref, aw_ref, ab_ref, f_ref, a_ref):
    f = jnp.dot(x_ref[...], w_ref[...], preferred_element_type=jnp.float32,
                precision=lax.Precision.HIGHEST) + b_ref[...]
    f_ref[...] = f
    a_ref[...] = jnp.dot(f, aw_ref[...], preferred_element_type=jnp.float32,
                         precision=lax.Precision.HIGHEST) + ab_ref[...]


def _dense(x, wt, b2, awt, ab2):
    n, din = x.shape
    p = wt.shape[1]
    na = awt.shape[1]
    return pl.pallas_call(
        _dense_body,
        grid=(n // BN,),
        in_specs=[pl.BlockSpec((BN, din), lambda i: (i, 0)),
                  pl.BlockSpec((din, p), lambda i: (0, 0)),
                  pl.BlockSpec((1, p), lambda i: (0, 0)),
                  pl.BlockSpec((p, na), lambda i: (0, 0)),
                  pl.BlockSpec((1, na), lambda i: (0, 0))],
        out_specs=[pl.BlockSpec((BN, p), lambda i: (i, 0)),
                   pl.BlockSpec((BN, na), lambda i: (i, 0))],
        out_shape=[jax.ShapeDtypeStruct((n, p), jnp.float32),
                   jax.ShapeDtypeStruct((n, na), jnp.float32)],
    )(x, wt, b2, awt, ab2)


ZR = 25
NBUF = 2


@functools.lru_cache(maxsize=None)
def _edge_kernel(P, heads=1):
    vpr = P // LN
    half = P // 2
    use_fsh = P <= 48 and heads == 1
    mesh = plsc.VectorSubcoreMesh(core_axis_name="c", subcore_axis_name="s")

    @functools.partial(
        pl.kernel,
        out_type=jax.ShapeDtypeStruct((NC, N, P), jnp.float32),
        mesh=mesh,
        compiler_params=pltpu.CompilerParams(use_tc_tiling_on_sc=False,
                                             needs_layout_passes=False),
        scratch_types=[
            pltpu.VMEM_SHARED((N, P), jnp.float32),
            pltpu.VMEM_SHARED((N if use_fsh else 1, P), jnp.float32),
            pltpu.VMEM((ZR, P), jnp.float32),
            pltpu.VMEM((2 * heads * N,), jnp.float32),
            pltpu.VMEM((LN,), jnp.float32),
            pltpu.VMEM((NCHUNK, CH), jnp.int32),
            pltpu.VMEM((NCHUNK, CH), jnp.int32),
        ] + [pltpu.VMEM((CH,), jnp.float32) for _ in range(heads)]
          + [pltpu.VMEM((CH, P), jnp.float32) for _ in range(NBUF)]
          + [pltpu.SemaphoreType.DMA for _ in range(NBUF + 1)],
    )
    def k(f_hbm, a12_hbm, row_hbm, col_hbm, gm_hbm, u_out,
          u_sh, f_sh, z_v, a12_v, gm_v, ridx, cidx, *tail):
        if heads == 1:
            (ex0_v, rows0, rows1, g0, g1, isem) = tail
        else:
            (ex0_v, ex1_v, rows0, rows1, g0, g1, isem) = tail
        cid = lax.axis_index("c")
        sid = lax.axis_index("s")
        wid = sid * NC + cid

        def zrow(r, carry):
            for j in range(vpr):
                z_v[r, pl.ds(j * LN, LN)] = jnp.zeros((LN,), jnp.float32)
            return carry
        lax.fori_loop(0, ZR, zrow, 0)

        cp_r = pltpu.async_copy(row_hbm.at[wid], ridx, isem)
        cp_c = pltpu.async_copy(col_hbm.at[wid], cidx, isem)
        if use_fsh:
            cp_f = pltpu.async_copy(f_hbm.at[pl.ds(sid * RPT, RPT)],
                                    f_sh.at[pl.ds(sid * RPT, RPT)], isem)
        f_tab = f_sh if use_fsh else f_hbm
        for t in range(RPT // ZR):
            pltpu.sync_copy(z_v, u_sh.at[pl.ds(sid * RPT + t * ZR, ZR)])
        pltpu.sync_copy(a12_hbm, a12_v)
        pltpu.sync_copy(gm_hbm, gm_v)
        cp_r.wait()
        cp_c.wait()
        if use_fsh:
            cp_f.wait()
        plsc.subcore_barrier()

        gmv = gm_v[...]
        rows = (rows0, rows1)
        gsems = (g0, g1)

        for b in range(NBUF):
            pltpu.async_copy(f_tab.at[cidx.at[b]], rows[b], gsems[b])

        def process(c, b):
            rv = rows[b]
            gs = gsems[b]
            pltpu.make_async_copy(f_tab.at[cidx.at[c]], rv, gs).wait()
            st = 2 * heads
            for g in range(CH // LN):
                r16 = ridx[c, pl.ds(g * LN, LN)]
                c16 = cidx[c, pl.ds(g * LN, LN)]
                a1 = plsc.load_gather(a12_v, [r16 * st])
                a2 = plsc.load_gather(a12_v, [c16 * st + 1])
                v = a1 + a2
                v = jnp.where(v >= 0.0, v, 0.01 * v)
                ex0_v[pl.ds(g * LN, LN)] = jnp.exp(v - gmv)
                if heads == 2:
                    b1 = plsc.load_gather(a12_v, [r16 * st + 2])
                    b2 = plsc.load_gather(a12_v, [c16 * st + 3])
                    w = b1 + b2
                    w = jnp.where(w >= 0.0, w, 0.01 * w)
                    ex1_v[pl.ds(g * LN, LN)] = jnp.exp(w - gmv)
            lane = lax.iota(jnp.int32, LN)
            for g in range(CH // LN):
                ev0 = ex0_v[pl.ds(g * LN, LN)]
                ev1 = ex1_v[pl.ds(g * LN, LN)] if heads == 2 else None
                for rr in range(LN):
                    r = g * LN + rr
                    e0 = ev0[rr]
                    for j in range(vpr):
                        lo = j * LN
                        if heads == 1 or lo + LN <= half:
                            e = e0
                        elif lo >= half:
                            e = ev1[rr]
                        else:
                            e = jnp.where(lane < (half - lo), e0, ev1[rr])
                        rv[r, pl.ds(lo, LN)] = rv[r, pl.ds(lo, LN)] * e
            pltpu.sync_copy(rv, u_sh.at[ridx.at[c]], add=True)

            @pl.when(c + NBUF < NCHUNK)
            def _next():
                pltpu.async_copy(f_tab.at[cidx.at[c + NBUF]], rv, gs)

        def quad(t, carry):
            for b in range(NBUF):
                process(NBUF * t + b, b)
            return carry
        lax.fori_loop(0, NCHUNK // NBUF, quad, 0)
        process(NCHUNK - 1, 0)

        plsc.subcore_barrier()
        pltpu.sync_copy(u_sh.at[pl.ds(sid * RPT, RPT)],
                        u_out.at[cid, pl.ds(sid * RPT, RPT)])

    return k


def _final_body(h0_ref, h1_ref, h2_ref, b0_ref, b1_ref, b2_ref,
                wc_ref, bc_ref, out_ref, acc, c0, c1, c2):
    i = pl.program_id(0)

    @pl.when(i == 0)
    def _init():
        acc[...] = jnp.zeros_like(acc)
        c0[...] = jnp.zeros_like(c0)
        c1[...] = jnp.zeros_like(c1)
        c2[...] = jnp.zeros_like(c2)

    iota = lax.broadcasted_iota(jnp.int32, (G, BN), 0)
    for lvl, (h_ref, b_ref, c_ref) in enumerate(
            ((h0_ref, b0_ref, c0), (h1_ref, b1_ref, c1), (h2_ref, b2_ref, c2))):
        m = (b_ref[0] == iota).astype(jnp.float32)
        acc[:, lvl * OUT:(lvl + 1) * OUT] += jnp.dot(
            m, h_ref[...], preferred_element_type=jnp.float32,
            precision=lax.Precision.HIGHEST)
        c_ref[...] += jnp.broadcast_to(
            jnp.sum(m, axis=1, keepdims=True), c_ref.shape)

    @pl.when(i == NB - 1)
    def _fin():
        cnt0 = jnp.maximum(c0[:, 0:1], 1.0)
        cnt1 = jnp.maximum(c1[:, 0:1], 1.0)
        cnt2 = jnp.maximum(c2[:, 0:1], 1.0)
        x = jnp.concatenate([acc[:, 0:OUT] / cnt0,
                             acc[:, OUT:2 * OUT] / cnt1,
                             acc[:, 2 * OUT:3 * OUT] / cnt2], axis=1)
        logits = jnp.dot(x, wc_ref[...], preferred_element_type=jnp.float32,
                         precision=lax.Precision.HIGHEST) + bc_ref[...]
        mx = jnp.max(logits, axis=1, keepdims=True)
        ez = jnp.exp(logits - mx)
        out_ref[...] = ez / jnp.sum(ez, axis=1, keepdims=True)


def _final(h0, h1, h2, b0, b1, b2, wc, bc):
    b3 = [b.reshape(NB, 1, BN) for b in (b0, b1, b2)]
    return pl.pallas_call(
        _final_body,
        grid=(NB,),
        in_specs=[pl.BlockSpec((BN, OUT), lambda i: (i, 0)),
                  pl.BlockSpec((BN, OUT), lambda i: (i, 0)),
                  pl.BlockSpec((BN, OUT), lambda i: (i, 0)),
                  pl.BlockSpec((1, 1, BN), lambda i: (i, 0, 0)),
                  pl.BlockSpec((1, 1, BN), lambda i: (i, 0, 0)),
                  pl.BlockSpec((1, 1, BN), lambda i: (i, 0, 0)),
                  pl.BlockSpec((3 * OUT, OUT), lambda i: (0, 0)),
                  pl.BlockSpec((1, OUT), lambda i: (0, 0))],
        out_specs=pl.BlockSpec((G, OUT), lambda i: (0, 0)),
        out_shape=jax.ShapeDtypeStruct((G, OUT), jnp.float32),
        scratch_shapes=[pltpu.VMEM((G, 3 * OUT), jnp.float32),
                        pltpu.VMEM((G, 128), jnp.float32),
                        pltpu.VMEM((G, 128), jnp.float32),
                        pltpu.VMEM((G, 128), jnp.float32)],
    )(h0, h1, h2, *b3, wc, bc)


def _pad_params(p, din, d, P):
    wt = jnp.zeros((din, P), jnp.float32).at[:, :d].set(p["W"].T)
    b2 = jnp.zeros((1, P), jnp.float32).at[0, :d].set(p["b"]).at[0, d].set(1.0)
    awt = jnp.zeros((P, 2), jnp.float32)
    awt = awt.at[:d, 0].set(p["a1w"][0]).at[:d, 1].set(p["a2w"][0])
    ab2 = jnp.concatenate([p["a1b"], p["a2b"]]).reshape(1, 2)
    return wt, b2, awt, ab2


def _sat(feats, row, col, p, din, d, P):
    wt, b2, awt, ab2 = _pad_params(p, din, d, P)
    f_aug, a12 = _dense(feats, wt, b2, awt, ab2)
    amax = jnp.max(a12, axis=0)
    gm = jnp.maximum(amax[0] + amax[1], 0.0)
    gm_arr = jnp.full((LN,), gm, jnp.float32)
    u = _edge_kernel(P)(f_aug, a12.reshape(-1), row, col, gm_arr)
    us = u[0] + u[1]
    return us[:, :d] / jnp.maximum(us[:, d:d + 1], 1e-30)


def _sat_cmb(feats, row3, col3, p0, p1, din, d, P):
    half = P // 2
    wt = (jnp.zeros((din, P), jnp.float32)
          .at[:, :d].set(p0["W"].T).at[:, half:half + d].set(p1["W"].T))
    b2 = (jnp.zeros((1, P), jnp.float32)
          .at[0, :d].set(p0["b"]).at[0, d].set(1.0)
          .at[0, half:half + d].set(p1["b"]).at[0, half + d].set(1.0))
    awt = (jnp.zeros((P, 4), jnp.float32)
           .at[:d, 0].set(p0["a1w"][0]).at[:d, 1].set(p0["a2w"][0])
           .at[half:half + d, 2].set(p1["a1w"][0])
           .at[half:half + d, 3].set(p1["a2w"][0]))
    ab2 = jnp.concatenate([p0["a1b"], p0["a2b"], p1["a1b"], p1["a2b"]]).reshape(1, 4)
    f_aug, a12 = _dense(feats, wt, b2, awt, ab2)
    amax = jnp.max(a12, axis=0)
    gm = jnp.maximum(jnp.maximum(amax[0] + amax[1], amax[2] + amax[3]), 0.0)
    gm_arr = jnp.full((LN,), gm, jnp.float32)
    u = _edge_kernel(P, 2)(f_aug, a12.reshape(-1), row3, col3, gm_arr)
    us = u[0] + u[1]
    o0 = us[:, :d] / jnp.maximum(us[:, d:d + 1], 1e-30)
    o1 = us[:, half:half + d] / jnp.maximum(us[:, half + d:half + d + 1], 1e-30)
    return o0, o1


def _level(X, idxs, lvl, params):
    d1 = FS // KH
    shared = idxs[0] is idxs[1]
    if shared:
        row3, col3 = idxs[0]
        h1 = jax.nn.relu(jnp.concatenate(
            _sat_cmb(X, row3, col3, params["l%ss1h0" % lvl],
                     params["l%ss1h1" % lvl], DF, d1, 48), axis=1))
        h2 = jax.nn.relu(jnp.concatenate(
            _sat_cmb(h1, row3, col3, params["l%ss2h0" % lvl],
                     params["l%ss2h1" % lvl], FS, d1, 48), axis=1))
    else:
        h1 = jax.nn.relu(jnp.concatenate(
            [_sat(X, idxs[h][0], idxs[h][1], params["l%ss1h%d" % (lvl, h)],
                  DF, d1, 32) for h in range(KH)], axis=1))
        h2 = jax.nn.relu(jnp.concatenate(
            [_sat(h1, idxs[h][0], idxs[h][1], params["l%ss2h%d" % (lvl, h)],
                  FS, d1, 32) for h in range(KH)], axis=1))
    h3 = sum(_sat(h2, idxs[h][0], idxs[h][1], params["l%ss3h%d" % (lvl, h)],
                  FS, OUT, 80) for h in range(KH)) / 2.0
    return h3


def kernel(params, X0, X1, X2, L0, L1u, L1d, L2, batch0, batch1, batch2):
    def _sh(a):
        return a.reshape(NW, NCHUNK, CH)
    e0 = (_sh(L0[0]), _sh(L0[1]))
    e1u = (_sh(L1u[0]), _sh(L1u[1]))
    e1d = (_sh(L1d[0]), _sh(L1d[1]))
    e2 = (_sh(L2[0]), _sh(L2[1]))
    h3_0 = _level(X0, [e0, e0], "0", params)
    h3_1 = _level(X1, [e1u, e1d], "1", params)
    h3_2 = _level(X2, [e2, e2], "2", params)
    wc = params["out"]["W"].T
    bc = params["out"]["b"].reshape(1, OUT)
    return _final(h3_0, h3_1, h3_2, batch0, batch1, batch2, wc, bc)


# speedup vs baseline: 1.1354x; 1.0270x over previous
import functools

import jax
import jax.numpy as jnp
from jax import lax
from jax.experimental import pallas as pl
from jax.experimental.pallas import tpu as pltpu
from jax.experimental.pallas import tpu_sc as plsc

N = 10000
E = 320000
DF = 128
FS = 40
KH = 2
OUT = 64
G = 16

NC = 2
NS = 16
LN = 16
NW = NC * NS
EPW = E // NW
CH = 80
NCHUNK = EPW // CH
RPT = N // NS

BN = 2000
NB = N // BN


def _dense_body(x_ref, w_ref, b_ref, aw_ref, ab_ref, f_ref, a_ref):
    f = jnp.dot(x_ref[...], w_ref[...], preferred_element_type=jnp.float32,
                precision=lax.Precision.HIGHEST) + b_ref[...]
    f_ref[...] = f
    a_ref[...] = jnp.dot(f, aw_ref[...], preferred_element_type=jnp.float32,
                         precision=lax.Precision.HIGHEST) + ab_ref[...]


def _dense(x, wt, b2, awt, ab2):
    n, din = x.shape
    p = wt.shape[1]
    na = awt.shape[1]
    return pl.pallas_call(
        _dense_body,
        grid=(n // BN,),
        in_specs=[pl.BlockSpec((BN, din), lambda i: (i, 0)),
                  pl.BlockSpec((din, p), lambda i: (0, 0)),
                  pl.BlockSpec((1, p), lambda i: (0, 0)),
                  pl.BlockSpec((p, na), lambda i: (0, 0)),
                  pl.BlockSpec((1, na), lambda i: (0, 0))],
        out_specs=[pl.BlockSpec((BN, p), lambda i: (i, 0)),
                   pl.BlockSpec((BN, na), lambda i: (i, 0))],
        out_shape=[jax.ShapeDtypeStruct((n, p), jnp.float32),
                   jax.ShapeDtypeStruct((n, na), jnp.float32)],
    )(x, wt, b2, awt, ab2)


ZR = 25
NBUF = 2


@functools.lru_cache(maxsize=None)
def _edge_kernel(P, heads=1, bf=False):
    vpr = P // LN
    half = P // 2
    pbf = 96
    use_fsh = P <= 48 and heads == 1
    mesh = plsc.VectorSubcoreMesh(core_axis_name="c", subcore_axis_name="s")

    @functools.partial(
        pl.kernel,
        out_type=jax.ShapeDtypeStruct((NC, N, P), jnp.float32),
        mesh=mesh,
        compiler_params=pltpu.CompilerParams(use_tc_tiling_on_sc=False,
                                             needs_layout_passes=False),
        scratch_types=[
            pltpu.VMEM_SHARED((N, P), jnp.float32),
            pltpu.VMEM_SHARED((N if use_fsh else 1, P), jnp.float32),
            pltpu.VMEM((ZR, P), jnp.float32),
            pltpu.VMEM((2 * heads * N,), jnp.float32),
            pltpu.VMEM((LN,), jnp.float32),
            pltpu.VMEM((NCHUNK, CH), jnp.int32),
            pltpu.VMEM((NCHUNK, CH), jnp.int32),
        ] + [pltpu.VMEM((CH,), jnp.float32) for _ in range(heads)]
          + ([pltpu.VMEM((CH, pbf), jnp.bfloat16) for _ in range(NBUF)]
             + [pltpu.VMEM((CH, P), jnp.float32)] if bf else
             [pltpu.VMEM((CH, P), jnp.float32) for _ in range(NBUF)])
          + [pltpu.SemaphoreType.DMA for _ in range(NBUF + 1)],
    )
    def k(f_hbm, a12_hbm, row_hbm, col_hbm, gm_hbm, u_out,
          u_sh, f_sh, z_v, a12_v, gm_v, ridx, cidx, *tail):
        rvf = None
        if heads == 1 and bf:
            (ex0_v, rows0, rows1, rvf, g0, g1, isem) = tail
        elif heads == 1:
            (ex0_v, rows0, rows1, g0, g1, isem) = tail
        else:
            (ex0_v, ex1_v, rows0, rows1, g0, g1, isem) = tail
        cid = lax.axis_index("c")
        sid = lax.axis_index("s")
        wid = sid * NC + cid

        def zrow(r, carry):
            for j in range(vpr):
                z_v[r, pl.ds(j * LN, LN)] = jnp.zeros((LN,), jnp.float32)
            return carry
        lax.fori_loop(0, ZR, zrow, 0)

        cp_r = pltpu.async_copy(row_hbm.at[wid], ridx, isem)
        cp_c = pltpu.async_copy(col_hbm.at[wid], cidx, isem)
        if use_fsh:
            cp_f = pltpu.async_copy(f_hbm.at[pl.ds(sid * RPT, RPT)],
                                    f_sh.at[pl.ds(sid * RPT, RPT)], isem)
        f_tab = f_sh if use_fsh else f_hbm
        for t in range(RPT // ZR):
            pltpu.sync_copy(z_v, u_sh.at[pl.ds(sid * RPT + t * ZR, ZR)])
        pltpu.sync_copy(a12_hbm, a12_v)
        pltpu.sync_copy(gm_hbm, gm_v)
        cp_r.wait()
        cp_c.wait()
        if use_fsh:
            cp_f.wait()
        plsc.subcore_barrier()

        gmv = gm_v[...]
        rows = (rows0, rows1)
        gsems = (g0, g1)

        for b in range(NBUF):
            pltpu.async_copy(f_tab.at[cidx.at[b]], rows[b], gsems[b])

        def process(c, b):
            rv = rows[b]
            gs = gsems[b]
            pltpu.make_async_copy(f_tab.at[cidx.at[c]], rv, gs).wait()
            st = 2 * heads
            for g in range(CH // LN):
                r16 = ridx[c, pl.ds(g * LN, LN)]
                c16 = cidx[c, pl.ds(g * LN, LN)]
                a1 = plsc.load_gather(a12_v, [r16 * st])
                a2 = plsc.load_gather(a12_v, [c16 * st + 1])
                v = a1 + a2
                v = jnp.where(v >= 0.0, v, 0.01 * v)
                ex0_v[pl.ds(g * LN, LN)] = jnp.exp(v - gmv)
                if heads == 2:
                    b1 = plsc.load_gather(a12_v, [r16 * st + 2])
                    b2 = plsc.load_gather(a12_v, [c16 * st + 3])
                    w = b1 + b2
                    w = jnp.where(w >= 0.0, w, 0.01 * w)
                    ex1_v[pl.ds(g * LN, LN)] = jnp.exp(w - gmv)
            lane = lax.iota(jnp.int32, LN)
            if bf:
                for g in range(CH // LN):
                    ev0 = ex0_v[pl.ds(g * LN, LN)]
                    for rr in range(LN):
                        r = g * LN + rr
                        e0 = ev0[rr]
                        for grp in range(P // 32 + 1):
                            pk = rv[r, pl.ds(grp * 32, 32)]
                            a, b2 = plsc.unpack(
                                pk, format=plsc.PackFormat.INTERLEAVED)
                            rvf[r, pl.ds(grp * 32, LN)] = a * e0
                            if grp * 32 + LN < P:
                                rvf[r, pl.ds(grp * 32 + LN, LN)] = b2 * e0
                pltpu.sync_copy(rvf, u_sh.at[ridx.at[c]], add=True)
            else:
                for g in range(CH // LN):
                    ev0 = ex0_v[pl.ds(g * LN, LN)]
                    ev1 = ex1_v[pl.ds(g * LN, LN)] if heads == 2 else None
                    for rr in range(LN):
                        r = g * LN + rr
                        e0 = ev0[rr]
                        for j in range(vpr):
                            lo = j * LN
                            if heads == 1 or lo + LN <= half:
                                e = e0
                            elif lo >= half:
                                e = ev1[rr]
                            else:
                                e = jnp.where(lane < (half - lo), e0, ev1[rr])
                            rv[r, pl.ds(lo, LN)] = rv[r, pl.ds(lo, LN)] * e
                pltpu.sync_copy(rv, u_sh.at[ridx.at[c]], add=True)

            @pl.when(c + NBUF < NCHUNK)
            def _next():
                pltpu.async_copy(f_tab.at[cidx.at[c + NBUF]], rv, gs)

        def quad(t, carry):
            for b in range(NBUF):
                process(NBUF * t + b, b)
            return carry
        lax.fori_loop(0, NCHUNK // NBUF, quad, 0)
        process(NCHUNK - 1, 0)

        plsc.subcore_barrier()
        pltpu.sync_copy(u_sh.at[pl.ds(sid * RPT, RPT)],
                        u_out.at[cid, pl.ds(sid * RPT, RPT)])

    return k


def _final_body(h0_ref, h1_ref, h2_ref, b0_ref, b1_ref, b2_ref,
                wc_ref, bc_ref, out_ref, acc, c0, c1, c2):
    i = pl.program_id(0)

    @pl.when(i == 0)
    def _init():
        acc[...] = jnp.zeros_like(acc)
        c0[...] = jnp.zeros_like(c0)
        c1[...] = jnp.zeros_like(c1)
        c2[...] = jnp.zeros_like(c2)

    iota = lax.broadcasted_iota(jnp.int32, (G, BN), 0)
    for lvl, (h_ref, b_ref, c_ref) in enumerate(
            ((h0_ref, b0_ref, c0), (h1_ref, b1_ref, c1), (h2_ref, b2_ref, c2))):
        m = (b_ref[0] == iota).astype(jnp.float32)
        acc[:, lvl * OUT:(lvl + 1) * OUT] += jnp.dot(
            m, h_ref[...], preferred_element_type=jnp.float32,
            precision=lax.Precision.HIGHEST)
        c_ref[...] += jnp.broadcast_to(
            jnp.sum(m, axis=1, keepdims=True), c_ref.shape)

    @pl.when(i == NB - 1)
    def _fin():
        cnt0 = jnp.maximum(c0[:, 0:1], 1.0)
        cnt1 = jnp.maximum(c1[:, 0:1], 1.0)
        cnt2 = jnp.maximum(c2[:, 0:1], 1.0)
        x = jnp.concatenate([acc[:, 0:OUT] / cnt0,
                             acc[:, OUT:2 * OUT] / cnt1,
                             acc[:, 2 * OUT:3 * OUT] / cnt2], axis=1)
        logits = jnp.dot(x, wc_ref[...], preferred_element_type=jnp.float32,
                         precision=lax.Precision.HIGHEST) + bc_ref[...]
        mx = jnp.max(logits, axis=1, keepdims=True)
        ez = jnp.exp(logits - mx)
        out_ref[...] = ez / jnp.sum(ez, axis=1, keepdims=True)


def _final(h0, h1, h2, b0, b1, b2, wc, bc):
    b3 = [b.reshape(NB, 1, BN) for b in (b0, b1, b2)]
    return pl.pallas_call(
        _final_body,
        grid=(NB,),
        in_specs=[pl.BlockSpec((BN, OUT), lambda i: (i, 0)),
                  pl.BlockSpec((BN, OUT), lambda i: (i, 0)),
                  pl.BlockSpec((BN, OUT), lambda i: (i, 0)),
                  pl.BlockSpec((1, 1, BN), lambda i: (i, 0, 0)),
                  pl.BlockSpec((1, 1, BN), lambda i: (i, 0, 0)),
                  pl.BlockSpec((1, 1, BN), lambda i: (i, 0, 0)),
                  pl.BlockSpec((3 * OUT, OUT), lambda i: (0, 0)),
                  pl.BlockSpec((1, OUT), lambda i: (0, 0))],
        out_specs=pl.BlockSpec((G, OUT), lambda i: (0, 0)),
        out_shape=jax.ShapeDtypeStruct((G, OUT), jnp.float32),
        scratch_shapes=[pltpu.VMEM((G, 3 * OUT), jnp.float32),
                        pltpu.VMEM((G, 128), jnp.float32),
                        pltpu.VMEM((G, 128), jnp.float32),
                        pltpu.VMEM((G, 128), jnp.float32)],
    )(h0, h1, h2, *b3, wc, bc)


def _pad_params(p, din, d, P):
    wt = jnp.zeros((din, P), jnp.float32).at[:, :d].set(p["W"].T)
    b2 = jnp.zeros((1, P), jnp.float32).at[0, :d].set(p["b"]).at[0, d].set(1.0)
    awt = jnp.zeros((P, 2), jnp.float32)
    awt = awt.at[:d, 0].set(p["a1w"][0]).at[:d, 1].set(p["a2w"][0])
    ab2 = jnp.concatenate([p["a1b"], p["a2b"]]).reshape(1, 2)
    return wt, b2, awt, ab2


import numpy as _np

_BF_PERM = _np.concatenate(
    [32 * g + (_np.arange(32) // 2) + 16 * (_np.arange(32) % 2)
     for g in range(3)])


def _sat(feats, row, col, p, din, d, P, bf=False):
    wt, b2, awt, ab2 = _pad_params(p, din, d, P)
    f_aug, a12 = _dense(feats, wt, b2, awt, ab2)
    amax = jnp.max(a12, axis=0)
    gm = jnp.maximum(amax[0] + amax[1], 0.0)
    gm_arr = jnp.full((LN,), gm, jnp.float32)
    if bf:
        f96 = jnp.zeros((N, 96), jnp.float32).at[:, :P].set(f_aug)
        f_tab = f96[:, _BF_PERM].astype(jnp.bfloat16)
        u = _edge_kernel(P, 1, True)(f_tab, a12.reshape(-1), row, col, gm_arr)
    else:
        u = _edge_kernel(P)(f_aug, a12.reshape(-1), row, col, gm_arr)
    us = u[0] + u[1]
    return us[:, :d] / jnp.maximum(us[:, d:d + 1], 1e-30)


def _sat_cmb(feats, row3, col3, p0, p1, din, d, P):
    half = P // 2
    wt = (jnp.zeros((din, P), jnp.float32)
          .at[:, :d].set(p0["W"].T).at[:, half:half + d].set(p1["W"].T))
    b2 = (jnp.zeros((1, P), jnp.float32)
          .at[0, :d].set(p0["b"]).at[0, d].set(1.0)
          .at[0, half:half + d].set(p1["b"]).at[0, half + d].set(1.0))
    awt = (jnp.zeros((P, 4), jnp.float32)
           .at[:d, 0].set(p0["a1w"][0]).at[:d, 1].set(p0["a2w"][0])
           .at[half:half + d, 2].set(p1["a1w"][0])
           .at[half:half + d, 3].set(p1["a2w"][0]))
    ab2 = jnp.concatenate([p0["a1b"], p0["a2b"], p1["a1b"], p1["a2b"]]).reshape(1, 4)
    f_aug, a12 = _dense(feats, wt, b2, awt, ab2)
    amax = jnp.max(a12, axis=0)
    gm = jnp.maximum(jnp.maximum(amax[0] + amax[1], amax[2] + amax[3]), 0.0)
    gm_arr = jnp.full((LN,), gm, jnp.float32)
    u = _edge_kernel(P, 2)(f_aug, a12.reshape(-1), row3, col3, gm_arr)
    us = u[0] + u[1]
    o0 = us[:, :d] / jnp.maximum(us[:, d:d + 1], 1e-30)
    o1 = us[:, half:half + d] / jnp.maximum(us[:, half + d:half + d + 1], 1e-30)
    return o0, o1


def _level(X, idxs, lvl, params):
    d1 = FS // KH
    shared = idxs[0] is idxs[1]
    if shared:
        row3, col3 = idxs[0]
        h1 = jax.nn.relu(jnp.concatenate(
            _sat_cmb(X, row3, col3, params["l%ss1h0" % lvl],
                     params["l%ss1h1" % lvl], DF, d1, 48), axis=1))
        h2 = jax.nn.relu(jnp.concatenate(
            _sat_cmb(h1, row3, col3, params["l%ss2h0" % lvl],
                     params["l%ss2h1" % lvl], FS, d1, 48), axis=1))
    else:
        h1 = jax.nn.relu(jnp.concatenate(
            [_sat(X, idxs[h][0], idxs[h][1], params["l%ss1h%d" % (lvl, h)],
                  DF, d1, 32) for h in range(KH)], axis=1))
        h2 = jax.nn.relu(jnp.concatenate(
            [_sat(h1, idxs[h][0], idxs[h][1], params["l%ss2h%d" % (lvl, h)],
                  FS, d1, 32) for h in range(KH)], axis=1))
    h3 = sum(_sat(h2, idxs[h][0], idxs[h][1], params["l%ss3h%d" % (lvl, h)],
                  FS, OUT, 80, bf=True) for h in range(KH)) / 2.0
    return h3


def kernel(params, X0, X1, X2, L0, L1u, L1d, L2, batch0, batch1, batch2):
    def _sh(a):
        return a.reshape(NW, NCHUNK, CH)
    e0 = (_sh(L0[0]), _sh(L0[1]))
    e1u = (_sh(L1u[0]), _sh(L1u[1]))
    e1d = (_sh(L1d[0]), _sh(L1d[1]))
    e2 = (_sh(L2[0]), _sh(L2[1]))
    h3_0 = _level(X0, [e0, e0], "0", params)
    h3_1 = _level(X1, [e1u, e1d], "1", params)
    h3_2 = _level(X2, [e2, e2], "2", params)
    wc = params["out"]["W"].T
    bc = params["out"]["b"].reshape(1, OUT)
    return _final(h3_0, h3_1, h3_2, batch0, batch1, batch2, wc, bc)
